# baseline (speedup 1.0000x reference)
import functools

import jax
import jax.numpy as jnp
from jax import lax
from jax.experimental import pallas as pl
from jax.experimental.pallas import tpu as pltpu
from jax.experimental.pallas import tpu_sc as plsc


def _lse_body(x_ref, out_ref, *, nblocks):
    x = x_ref[...]
    m = jnp.max(x, axis=-1, keepdims=True)
    s = jnp.sum(jnp.exp(x - m), axis=-1, keepdims=True)
    lse = m + jnp.log(s)
    part = jnp.sum(lse, keepdims=True).reshape(1, 1)

    i = pl.program_id(0)

    @pl.when(i == 0)
    def _init():
        out_ref[...] = jnp.zeros((1, 1), jnp.float32)

    out_ref[...] += part


@functools.partial(jax.jit, static_argnames=("block_tokens",))
def _lse_sum(logits_flat, block_tokens):
    num_tokens, vocab = logits_flat.shape
    nblocks = num_tokens // block_tokens
    out = pl.pallas_call(
        functools.partial(_lse_body, nblocks=nblocks),
        grid=(nblocks,),
        in_specs=[pl.BlockSpec((block_tokens, vocab), lambda i: (i, 0))],
        out_specs=pl.BlockSpec((1, 1), lambda i: (0, 0)),
        out_shape=jax.ShapeDtypeStruct((1, 1), jnp.float32),
    )(logits_flat)
    return out[0, 0]


def _sc_tgt_sum(logits_1d, targets, num_tokens, vocab):
    n_workers = 32
    per_w = num_tokens // n_workers
    n_chunks = per_w // 16
    mesh = plsc.VectorSubcoreMesh(core_axis_name="c", subcore_axis_name="s")

    @functools.partial(
        pl.kernel,
        mesh=mesh,
        out_type=jax.ShapeDtypeStruct((2, 16), jnp.float32),
        scratch_types=[
            pltpu.VMEM((per_w,), jnp.int32),
            pltpu.VMEM((per_w,), jnp.float32),
            pltpu.VMEM((16,), jnp.float32),
            pltpu.VMEM((16, 16), jnp.float32),
            pltpu.VMEM_SHARED((16, 16), jnp.float32),
            pltpu.SemaphoreType.DMA,
        ],
    )
    def k(flat_hbm, tgt_hbm, out_hbm, idx_v, rows_v, acc_v, all_v, shared, sem):
        c = lax.axis_index("c")
        s = lax.axis_index("s")
        wid = s * 2 + c
        base = wid * per_w
        pltpu.sync_copy(tgt_hbm.at[pl.ds(base, per_w)], idx_v)
        for i in range(n_chunks):
            t = idx_v[pl.ds(i * 16, 16)]
            row = base + i * 16 + lax.iota(jnp.int32, 16)
            idx_v[pl.ds(i * 16, 16)] = row * vocab + t
        pltpu.async_copy(flat_hbm.at[idx_v], rows_v, sem).wait()
        acc = jnp.zeros((16,), jnp.float32)
        for i in range(n_chunks):
            acc = acc + rows_v[pl.ds(i * 16, 16)]
        acc_v[...] = acc
        pltpu.sync_copy(acc_v, shared.at[s])
        plsc.subcore_barrier()

        @pl.when(s == 0)
        def _():
            pltpu.sync_copy(shared, all_v)
            tot = jnp.zeros((16,), jnp.float32)
            for r in range(16):
                tot = tot + all_v[r]
            acc_v[...] = tot
            pltpu.sync_copy(acc_v, out_hbm.at[c])

    return k(logits_1d, targets)


def kernel(logits, targets):
    vocab = logits.shape[-1]
    logits_flat = logits.reshape(-1, vocab)
    num_tokens = logits_flat.shape[0]
    targets_flat = targets.reshape(-1).astype(jnp.int32)

    tgt_partials = _sc_tgt_sum(
        logits_flat.reshape(-1), targets_flat, num_tokens, vocab
    )
    lse_sum = _lse_sum(logits_flat, 128)
    tgt_sum = jnp.sum(tgt_partials)
    return (lse_sum - tgt_sum) * (1.0 / num_tokens)
